# baseline (speedup 1.0000x reference)
import functools

import jax
import jax.numpy as jnp
from jax import lax
from jax.experimental import pallas as pl
from jax.experimental.pallas import tpu as pltpu
from jax.experimental.pallas import tpu_sc as plsc


def _xnorm_body(x_ref, xnb_ref, x2_ref):
    xb = x_ref[0]
    mu = jnp.mean(xb, axis=0, keepdims=True)
    std = jnp.std(xb, axis=0, keepdims=True, ddof=1)
    xn = (xb - mu) / (std + 1e-5)
    xnb_ref[0] = xn.astype(jnp.bfloat16)
    x2_ref[0] = jnp.sum(xn * xn, axis=1, keepdims=True)


def _xnorm(x):
    N, T, D = x.shape
    return pl.pallas_call(
        _xnorm_body,
        grid=(N,),
        in_specs=[pl.BlockSpec((1, T, D), lambda n: (n, 0, 0))],
        out_specs=[
            pl.BlockSpec((1, T, D), lambda n: (n, 0, 0)),
            pl.BlockSpec((1, T, 1), lambda n: (n, 0, 0)),
        ],
        out_shape=[
            jax.ShapeDtypeStruct((N, T, D), jnp.bfloat16),
            jax.ShapeDtypeStruct((N, T, 1), jnp.float32),
        ],
    )(x)


def _embnorm_body(et_ref, enb_ref, e2_ref):
    et = et_ref[...]
    nrm = jnp.sqrt(jnp.sum(et * et, axis=0, keepdims=True))
    en = et / (nrm + 1e-4)
    e2_ref[...] = jnp.sum(en * en, axis=0, keepdims=True)
    enb_ref[...] = en.astype(jnp.bfloat16)


def _embnorm(emb_t):
    D, M = emb_t.shape
    BM = 1024
    return pl.pallas_call(
        _embnorm_body,
        grid=(M // BM,),
        in_specs=[pl.BlockSpec((D, BM), lambda m: (0, m))],
        out_specs=[
            pl.BlockSpec((D, BM), lambda m: (0, m)),
            pl.BlockSpec((1, BM), lambda m: (0, m)),
        ],
        out_shape=[
            jax.ShapeDtypeStruct((D, M), jnp.bfloat16),
            jax.ShapeDtypeStruct((1, M), jnp.float32),
        ],
    )(emb_t)


def _dist_body(xnb_ref, x2_ref, enb_ref, e2_ref, idx_ref, mv_s, mi_s,
               *, T, BM, M):
    m = pl.program_id(1)
    nm = pl.num_programs(1)

    s = lax.dot_general(xnb_ref[0], enb_ref[:, pl.ds(m * BM, BM)],
                        (((1,), (0,)), ((), ())),
                        preferred_element_type=jnp.float32)
    dist = (e2_ref[:, pl.ds(m * BM, BM)] + x2_ref[0]) - 2.0 * s
    rowmin = jnp.min(dist, axis=1, keepdims=True)
    colf = lax.broadcasted_iota(jnp.int32, (1, BM), 1).astype(jnp.float32)
    candf = jnp.where(dist == rowmin, colf, jnp.float32(BM))
    barg = jnp.min(candf, axis=1, keepdims=True).astype(jnp.int32) + m * BM
    init = m == 0
    prev = jnp.where(init, jnp.inf, mv_s[...])
    previ = jnp.where(init, 0, mi_s[...])
    better = rowmin < prev
    mi_s[...] = jnp.where(better, barg, previ)
    mv_s[...] = jnp.where(better, rowmin, prev)
    idx_ref[0] = mi_s[...]


def _dist_argmin(xnb, x2, enb, e2):
    N, T, D = xnb.shape
    M = enb.shape[1]
    BM = 1024
    return pl.pallas_call(
        functools.partial(_dist_body, T=T, BM=BM, M=M),
        grid=(N, M // BM),
        in_specs=[
            pl.BlockSpec((1, T, D), lambda n, m: (n, 0, 0)),
            pl.BlockSpec((1, T, 1), lambda n, m: (n, 0, 0)),
            pl.BlockSpec((D, M), lambda n, m: (0, 0)),
            pl.BlockSpec((1, M), lambda n, m: (0, 0)),
        ],
        out_specs=pl.BlockSpec((1, T, 1), lambda n, m: (n, 0, 0)),
        out_shape=jax.ShapeDtypeStruct((N, T, 1), jnp.int32),
        scratch_shapes=[
            pltpu.VMEM((T, 1), jnp.float32),
            pltpu.VMEM((T, 1), jnp.int32),
        ],
    )(xnb, x2, enb, e2)


def _sc_gather(table, idx_flat):
    M, D = table.shape
    B = idx_flat.shape[0]
    info = plsc.get_sparse_core_info()
    NC, NS = info.num_cores, info.num_subcores
    NW = NC * NS
    b_per_w = B // NW
    mesh = plsc.VectorSubcoreMesh(core_axis_name="c", subcore_axis_name="s")

    @functools.partial(
        pl.kernel, mesh=mesh,
        out_type=jax.ShapeDtypeStruct((B, D), jnp.float32),
        scratch_types=[
            pltpu.VMEM((b_per_w,), jnp.int32),
            pltpu.VMEM((b_per_w, D), jnp.float32),
            pltpu.SemaphoreType.DMA,
        ],
    )
    def gather_k(table_hbm, idx_hbm, out_hbm, idx_v, rows_v, sem):
        wid = lax.axis_index("s") * NC + lax.axis_index("c")
        base = wid * b_per_w
        pltpu.sync_copy(idx_hbm.at[pl.ds(base, b_per_w)], idx_v)
        pltpu.async_copy(table_hbm.at[idx_v], rows_v, sem).wait()
        pltpu.sync_copy(rows_v, out_hbm.at[pl.ds(base, b_per_w)])

    return gather_k(table, idx_flat)


def _stats_body(xnb_ref, q_ref, idx_ref, qout_ref, loss_ref, perp_ref,
                sum_s, cnt_s, *, N, T, D, M):
    n = pl.program_id(0)
    MH, ML = 64, 128

    xn = xnb_ref[0].astype(jnp.float32)
    q = q_ref[0]
    d = xn - q
    part = jnp.sum(d * d, axis=(0, 1), keepdims=True)
    sum_s[...] = jnp.where(n == 0, part, sum_s[...] + part)
    t = xn + (q - xn)
    qout_ref[0] = (t + q) / 2.0
    idxb = idx_ref[0]
    hi = idxb // ML
    lo = idxb - hi * ML
    ohh = (hi == lax.broadcasted_iota(jnp.int32, (T, MH), 1)).astype(jnp.float32)
    ohl = (lo == lax.broadcasted_iota(jnp.int32, (T, ML), 1)).astype(jnp.float32)
    part_cnt = lax.dot_general(ohh, ohl, (((0,), (0,)), ((), ())),
                               preferred_element_type=jnp.float32)
    cnt_s[...] = jnp.where(n == 0, part_cnt, cnt_s[...] + part_cnt)

    @pl.when(n == pl.num_programs(0) - 1)
    def _fin():
        loss_ref[...] = sum_s[...] / (N * T * D)
        p = cnt_s[...] / (N * T)
        ent = jnp.sum(p * jnp.log(p + 1e-10), axis=(0, 1), keepdims=True)
        perp_ref[...] = jnp.exp(-ent)


def _stats(xnb, q, idx, M):
    N, T, D = xnb.shape
    return pl.pallas_call(
        functools.partial(_stats_body, N=N, T=T, D=D, M=M),
        grid=(N,),
        in_specs=[
            pl.BlockSpec((1, T, D), lambda n: (n, 0, 0)),
            pl.BlockSpec((1, T, D), lambda n: (n, 0, 0)),
            pl.BlockSpec((1, T, 1), lambda n: (n, 0, 0)),
        ],
        out_specs=[
            pl.BlockSpec((1, T, D), lambda n: (n, 0, 0)),
            pl.BlockSpec((1, 1), lambda n: (0, 0)),
            pl.BlockSpec((1, 1), lambda n: (0, 0)),
        ],
        out_shape=[
            jax.ShapeDtypeStruct((N, T, D), jnp.float32),
            jax.ShapeDtypeStruct((1, 1), jnp.float32),
            jax.ShapeDtypeStruct((1, 1), jnp.float32),
        ],
        scratch_shapes=[
            pltpu.VMEM((1, 1), jnp.float32),
            pltpu.VMEM((64, 128), jnp.float32),
        ],
    )(xnb, q, idx)


def kernel(x, embedding):
    N, T, D = x.shape
    xnb, x2 = _xnorm(x)
    enb, e2 = _embnorm(embedding.T)
    idx = _dist_argmin(xnb, x2, enb, e2)
    q = _sc_gather(embedding, idx.reshape(-1))
    qout, loss, perp = _stats(xnb, q.reshape(N, T, D), idx, embedding.shape[0])
    return qout, loss.reshape(()), perp.reshape(())
